# Initial kernel scaffold; baseline (speedup 1.0000x reference)
import jax
import jax.numpy as jnp
from jax.experimental import pallas as pl


def kernel(x, W, b, health):
    raise NotImplementedError("write your pallas kernel here")


# speedup vs baseline: 3.3035x; 3.3035x over previous
import jax
import jax.numpy as jnp
from jax.experimental import pallas as pl
from jax.experimental.pallas import tpu as pltpu

_B = 128
_D = 64
_N = 32768
_K = 32
_NT = 16
_TN = _N // _NT


def _scores_tile(x_ref, w_ref, b_ref, h_ref):
    xw = jax.lax.dot_general(
        x_ref[:], w_ref[:], (((1,), (1,)), ((), ())),
        preferred_element_type=jnp.float32)
    sig = 1.0 / (1.0 + jnp.exp(-h_ref[:]))
    return jnp.maximum(xw + b_ref[:], 0.0) * sig


def _select_kernel(x_ref, w_ref, b_ref, h_ref, tval_ref, tidx_ref, s_ref):
    i = pl.program_id(0)
    s_ref[:, pl.ds(i * _TN, _TN)] = _scores_tile(x_ref, w_ref, b_ref, h_ref)

    @pl.when(i == _NT - 1)
    def _():
        def body(k, p):
            cand = p | (1 << (30 - k))
            cf = jax.lax.bitcast_convert_type(cand, jnp.float32)
            cnt = jnp.sum((s_ref[:] >= cf).astype(jnp.int32), axis=1,
                          keepdims=True)
            return jnp.where(cnt >= _K, cand, p)

        t_int = jax.lax.fori_loop(0, 31, body, jnp.zeros((_B, 1), jnp.int32))
        t_f = jax.lax.bitcast_convert_type(t_int, jnp.float32)
        n_gt = jnp.sum((s_ref[:] > t_f).astype(jnp.int32), axis=1,
                       keepdims=True)
        quota = _K - n_gt

        def ibody(k, p):
            cand = p | (1 << (14 - k))
            col = jax.lax.broadcasted_iota(jnp.int32, (_B, _N), 1)
            g = jnp.sum(((s_ref[:] == t_f) & (col < cand)).astype(jnp.int32),
                        axis=1, keepdims=True)
            return jnp.where(g < quota, cand, p)

        idx_t = jax.lax.fori_loop(0, 15, ibody, jnp.zeros((_B, 1), jnp.int32))
        tval_ref[:] = jnp.broadcast_to(t_f, (_B, 128))
        tidx_ref[:] = jnp.broadcast_to(idx_t, (_B, 128))


def _mask_kernel(x_ref, w_ref, b_ref, h_ref, tval_ref, tidx_ref, o_ref):
    i = pl.program_id(0)
    s = _scores_tile(x_ref, w_ref, b_ref, h_ref)
    t = tval_ref[:, 0:1]
    it = tidx_ref[:, 0:1]
    col = i * _TN + jax.lax.broadcasted_iota(jnp.int32, (_B, _TN), 1)
    keep = (s > t) | ((s == t) & (col <= it))
    o_ref[:] = jnp.where(keep, s, 0.0)


def kernel(x, W, b, health):
    b2 = b.reshape(1, _N)
    h2 = health.reshape(1, _N)
    tval, tidx = pl.pallas_call(
        _select_kernel,
        grid=(_NT,),
        in_specs=[
            pl.BlockSpec((_B, _D), lambda i: (0, 0)),
            pl.BlockSpec((_TN, _D), lambda i: (i, 0)),
            pl.BlockSpec((1, _TN), lambda i: (0, i)),
            pl.BlockSpec((1, _TN), lambda i: (0, i)),
        ],
        out_specs=[
            pl.BlockSpec((_B, 128), lambda i: (0, 0)),
            pl.BlockSpec((_B, 128), lambda i: (0, 0)),
        ],
        out_shape=[
            jax.ShapeDtypeStruct((_B, 128), jnp.float32),
            jax.ShapeDtypeStruct((_B, 128), jnp.int32),
        ],
        scratch_shapes=[pltpu.VMEM((_B, _N), jnp.float32)],
    )(x, W, b2, h2)

    return pl.pallas_call(
        _mask_kernel,
        grid=(_NT,),
        in_specs=[
            pl.BlockSpec((_B, _D), lambda i: (0, 0)),
            pl.BlockSpec((_TN, _D), lambda i: (i, 0)),
            pl.BlockSpec((1, _TN), lambda i: (0, i)),
            pl.BlockSpec((1, _TN), lambda i: (0, i)),
            pl.BlockSpec((_B, 128), lambda i: (0, 0)),
            pl.BlockSpec((_B, 128), lambda i: (0, 0)),
        ],
        out_specs=pl.BlockSpec((_B, _TN), lambda i: (0, i)),
        out_shape=jax.ShapeDtypeStruct((_B, _N), jnp.float32),
    )(x, W, b2, h2, tval, tidx)
